# baseline (device time: 157192 ns/iter reference)
import jax
import jax.numpy as jnp
from jax import lax
from jax.experimental import pallas as pl
from jax.experimental.pallas import tpu as pltpu

N_DEV = 4
N_HOP = N_DEV - 1
T = 2


def kernel(x, w_mat):
    m_global, k_per = x.shape
    k_per2, n = w_mat.shape
    assert k_per == k_per2
    m_per = m_global // N_DEV
    n_half = n // 2
    tile = n_half // T

    def body(x_hbm, w_hbm, out_hbm, x_vmem, w_vmem, commR, commL,
             sR_sems, rR_sems, sL_sems, rL_sems, x_sems, w_sems, o_sems):
        my = lax.axis_index("i")
        left = (my - 1 + N_DEV) % N_DEV
        right = (my + 1) % N_DEV

        barrier_sem = pltpu.get_barrier_semaphore()
        for nbr in (left, right):
            pl.semaphore_signal(
                barrier_sem, inc=1,
                device_id=(nbr,), device_id_type=pl.DeviceIdType.MESH,
            )
        pl.semaphore_wait(barrier_sem, 2)

        def w_copy(half):
            return pltpu.make_async_copy(
                w_hbm.at[:, half * n_half:(half + 1) * n_half],
                w_vmem.at[:, half * n_half:(half + 1) * n_half],
                w_sems.at[half],
            )

        def x_copy(s, off):
            c_id = (my + off + N_DEV) % N_DEV
            return pltpu.make_async_copy(
                x_hbm.at[pl.ds(c_id * m_per, m_per), :],
                x_vmem.at[s],
                x_sems.at[s],
            )

        x_offs = (-1, 1, -2, 0)
        w_fetch = [w_copy(0)]
        x_fetch = [x_copy(0, x_offs[0])]
        w_fetch[0].start()
        x_fetch[0].start()

        def loc(slot, col0):
            return jnp.dot(
                x_vmem[slot], w_vmem[:, col0:col0 + tile],
                preferred_element_type=jnp.float32,
            )

        def mk(ring, h, t):
            comm, ssem, rsem, dst = {
                "R": (commR, sR_sems, rR_sems, right),
                "L": (commL, sL_sems, rL_sems, left),
            }[ring]
            return pltpu.make_async_remote_copy(
                src_ref=comm.at[h, t],
                dst_ref=comm.at[(h + 1) % N_HOP, t],
                send_sem=ssem.at[h, t],
                recv_sem=rsem.at[h, t],
                device_id=(dst,),
                device_id_type=pl.DeviceIdType.MESH,
            )

        descs = {}

        w_fetch[0].wait()
        x_fetch[0].wait()
        w_fetch.append(w_copy(1))
        x_fetch.append(x_copy(1, x_offs[1]))
        w_fetch[1].start()
        x_fetch[1].start()
        for t in range(T):
            commR[0, t] = loc(0, t * tile)
            d = descs[("R", 0, t)] = mk("R", 0, t)
            d.start()
        w_fetch[1].wait()
        x_fetch[1].wait()
        for s in (2, 3):
            x_fetch.append(x_copy(s, x_offs[s]))
            x_fetch[s].start()
        for t in range(T):
            commL[0, t] = loc(1, n_half + t * tile)
            d = descs[("L", 0, t)] = mk("L", 0, t)
            d.start()

        ring_slots = [(2, 2), (1, 0), (3, 3)]

        x_fetch[2].wait()
        out_dmas = []
        send_waited = set()
        for h in range(N_HOP):
            sR, sL = ring_slots[h]
            if h == N_HOP - 1:
                x_fetch[3].wait()
            for t in range(T):
                lR = loc(sR, t * tile)
                lL = loc(sL, n_half + t * tile)
                descs[("R", h, t)].wait_recv()
                descs[("L", h, t)].wait_recv()
                if h < N_HOP - 1:
                    if h == 0:
                        for ring in ("R", "L"):
                            descs[(ring, 0, t)].wait_send()
                            send_waited.add((ring, 0, t))
                    commR[h + 1, t] = commR[h + 1, t] + lR
                    commL[h + 1, t] = commL[h + 1, t] + lL
                    for ring in ("R", "L"):
                        d = descs[(ring, h + 1, t)] = mk(ring, h + 1, t)
                        d.start()
                else:
                    commR[0, t] = commR[0, t] + lR
                    commL[0, t] = commL[0, t] + lL
                    colR = t * tile
                    colL = n_half + t * tile
                    dR = pltpu.make_async_copy(
                        commR.at[0, t],
                        out_hbm.at[:, colR:colR + tile],
                        o_sems.at[t],
                    )
                    dL = pltpu.make_async_copy(
                        commL.at[0, t],
                        out_hbm.at[:, colL:colL + tile],
                        o_sems.at[T + t],
                    )
                    dR.start()
                    dL.start()
                    out_dmas.extend([dR, dL])

        for key, d in descs.items():
            if key not in send_waited:
                d.wait_send()
        for d in out_dmas:
            d.wait()

    return pl.pallas_call(
        body,
        out_shape=jax.ShapeDtypeStruct((m_per, n), jnp.float32),
        in_specs=[
            pl.BlockSpec(memory_space=pl.ANY),
            pl.BlockSpec(memory_space=pl.ANY),
        ],
        out_specs=pl.BlockSpec(memory_space=pl.ANY),
        scratch_shapes=[
            pltpu.VMEM((N_DEV, m_per, k_per), jnp.float32),
            pltpu.VMEM((k_per, n), jnp.float32),
            pltpu.VMEM((N_HOP, T, m_per, tile), jnp.float32),
            pltpu.VMEM((N_HOP, T, m_per, tile), jnp.float32),
            pltpu.SemaphoreType.DMA((N_HOP, T)),
            pltpu.SemaphoreType.DMA((N_HOP, T)),
            pltpu.SemaphoreType.DMA((N_HOP, T)),
            pltpu.SemaphoreType.DMA((N_HOP, T)),
            pltpu.SemaphoreType.DMA((N_DEV,)),
            pltpu.SemaphoreType.DMA((2,)),
            pltpu.SemaphoreType.DMA((2 * T,)),
        ],
        compiler_params=pltpu.CompilerParams(
            collective_id=0,
            vmem_limit_bytes=100 * 1024 * 1024,
        ),
    )(x, w_mat)


# device time: 156057 ns/iter; 1.0073x vs baseline; 1.0073x over previous
import jax
import jax.numpy as jnp
from jax import lax
from jax.experimental import pallas as pl
from jax.experimental.pallas import tpu as pltpu

N_DEV = 4
N_HOP = N_DEV - 1
T = 2


def kernel(x, w_mat):
    m_global, k_per = x.shape
    k_per2, n = w_mat.shape
    assert k_per == k_per2
    m_per = m_global // N_DEV
    n_half = n // 2
    tile = n_half // T

    def body(x_hbm, w_hbm, out_hbm, x_vmem, w_vmem, commR, commL,
             sR_sems, rR_sems, sL_sems, rL_sems, x_sems, w_sems, o_sems):
        my = lax.axis_index("i")
        left = (my - 1 + N_DEV) % N_DEV
        right = (my + 1) % N_DEV

        def w_copy(half):
            return pltpu.make_async_copy(
                w_hbm.at[:, half * n_half:(half + 1) * n_half],
                w_vmem.at[:, half * n_half:(half + 1) * n_half],
                w_sems.at[half],
            )

        def x_copy(s, off):
            c_id = (my + off + N_DEV) % N_DEV
            return pltpu.make_async_copy(
                x_hbm.at[pl.ds(c_id * m_per, m_per), :],
                x_vmem.at[s],
                x_sems.at[s],
            )

        x_offs = (-1, 1, -2, 0)
        w_fetch = [w_copy(0)]
        x_fetch = [x_copy(0, x_offs[0])]
        w_fetch[0].start()
        x_fetch[0].start()

        barrier_sem = pltpu.get_barrier_semaphore()
        for nbr in (left, right):
            pl.semaphore_signal(
                barrier_sem, inc=1,
                device_id=(nbr,), device_id_type=pl.DeviceIdType.MESH,
            )
        pl.semaphore_wait(barrier_sem, 2)

        def loc(slot, col0):
            return jnp.dot(
                x_vmem[slot], w_vmem[:, col0:col0 + tile],
                preferred_element_type=jnp.float32,
            )

        def mk(ring, h, t):
            comm, ssem, rsem, dst = {
                "R": (commR, sR_sems, rR_sems, right),
                "L": (commL, sL_sems, rL_sems, left),
            }[ring]
            return pltpu.make_async_remote_copy(
                src_ref=comm.at[h, t],
                dst_ref=comm.at[(h + 1) % N_HOP, t],
                send_sem=ssem.at[h, t],
                recv_sem=rsem.at[h, t],
                device_id=(dst,),
                device_id_type=pl.DeviceIdType.MESH,
            )

        descs = {}

        w_fetch[0].wait()
        x_fetch[0].wait()
        w_fetch.append(w_copy(1))
        x_fetch.append(x_copy(1, x_offs[1]))
        w_fetch[1].start()
        x_fetch[1].start()
        for t in range(T):
            commR[0, t] = loc(0, t * tile)
            d = descs[("R", 0, t)] = mk("R", 0, t)
            d.start()
        w_fetch[1].wait()
        x_fetch[1].wait()
        for s in (2, 3):
            x_fetch.append(x_copy(s, x_offs[s]))
            x_fetch[s].start()
        for t in range(T):
            commL[0, t] = loc(1, n_half + t * tile)
            d = descs[("L", 0, t)] = mk("L", 0, t)
            d.start()

        ring_slots = [(2, 2), (1, 0), (3, 3)]

        x_fetch[2].wait()
        out_dmas = []
        send_waited = set()
        for h in range(N_HOP):
            sR, sL = ring_slots[h]
            if h == N_HOP - 1:
                x_fetch[3].wait()
            for t in range(T):
                lR = loc(sR, t * tile)
                lL = loc(sL, n_half + t * tile)
                descs[("R", h, t)].wait_recv()
                descs[("L", h, t)].wait_recv()
                if h < N_HOP - 1:
                    if h == 0:
                        for ring in ("R", "L"):
                            descs[(ring, 0, t)].wait_send()
                            send_waited.add((ring, 0, t))
                    commR[h + 1, t] = commR[h + 1, t] + lR
                    commL[h + 1, t] = commL[h + 1, t] + lL
                    for ring in ("R", "L"):
                        d = descs[(ring, h + 1, t)] = mk(ring, h + 1, t)
                        d.start()
                else:
                    commR[0, t] = commR[0, t] + lR
                    commL[0, t] = commL[0, t] + lL
                    colR = t * tile
                    colL = n_half + t * tile
                    dR = pltpu.make_async_copy(
                        commR.at[0, t],
                        out_hbm.at[:, colR:colR + tile],
                        o_sems.at[t],
                    )
                    dL = pltpu.make_async_copy(
                        commL.at[0, t],
                        out_hbm.at[:, colL:colL + tile],
                        o_sems.at[T + t],
                    )
                    dR.start()
                    dL.start()
                    out_dmas.extend([dR, dL])

        for key, d in descs.items():
            if key not in send_waited:
                d.wait_send()
        for d in out_dmas:
            d.wait()

    return pl.pallas_call(
        body,
        out_shape=jax.ShapeDtypeStruct((m_per, n), jnp.float32),
        in_specs=[
            pl.BlockSpec(memory_space=pl.ANY),
            pl.BlockSpec(memory_space=pl.ANY),
        ],
        out_specs=pl.BlockSpec(memory_space=pl.ANY),
        scratch_shapes=[
            pltpu.VMEM((N_DEV, m_per, k_per), jnp.float32),
            pltpu.VMEM((k_per, n), jnp.float32),
            pltpu.VMEM((N_HOP, T, m_per, tile), jnp.float32),
            pltpu.VMEM((N_HOP, T, m_per, tile), jnp.float32),
            pltpu.SemaphoreType.DMA((N_HOP, T)),
            pltpu.SemaphoreType.DMA((N_HOP, T)),
            pltpu.SemaphoreType.DMA((N_HOP, T)),
            pltpu.SemaphoreType.DMA((N_HOP, T)),
            pltpu.SemaphoreType.DMA((N_DEV,)),
            pltpu.SemaphoreType.DMA((2,)),
            pltpu.SemaphoreType.DMA((2 * T,)),
        ],
        compiler_params=pltpu.CompilerParams(
            collective_id=0,
            vmem_limit_bytes=100 * 1024 * 1024,
        ),
    )(x, w_mat)
